# baseline (speedup 1.0000x reference)
import jax
import jax.numpy as jnp
from jax.experimental import pallas as pl
from jax.experimental.pallas import tpu as pltpu

BLOCK_C = 12800
N_BLOCKS = 8
DEPTH = 4
PAD_C = BLOCK_C * N_BLOCKS


def _sizes(n_classes):
    sizes = []
    for i in range(N_BLOCKS):
        sizes.append(min(BLOCK_C, n_classes - i * BLOCK_C))
    return sizes


def _cwr_body(x_ref, b_ref, w_hbm, o_ref, wbuf, sems):
    n_classes = w_hbm.shape[0]
    sizes = _sizes(n_classes)

    def issue(i):
        slot = i % DEPTH
        size = sizes[i]
        pltpu.make_async_copy(
            w_hbm.at[pl.ds(i * BLOCK_C, size)],
            wbuf.at[slot, pl.ds(0, size)],
            sems.at[slot],
        ).start()

    for i in range(min(DEPTH, N_BLOCKS)):
        issue(i)

    x = x_ref[...]
    for i in range(N_BLOCKS):
        slot = i % DEPTH
        size = sizes[i]
        pltpu.make_async_copy(
            w_hbm.at[pl.ds(i * BLOCK_C, size)],
            wbuf.at[slot, pl.ds(0, size)],
            sems.at[slot],
        ).wait()
        acc = jax.lax.dot_general(
            x,
            wbuf[slot],
            (((1,), (1,)), ((), ())),
            preferred_element_type=jnp.float32,
        )
        sl = slice(i * BLOCK_C, (i + 1) * BLOCK_C)
        o_ref[:, sl] = acc + b_ref[:, sl]
        if i + DEPTH < N_BLOCKS:
            issue(i + DEPTH)


@jax.jit
def kernel(x, weight, bias):
    n_classes, in_features = weight.shape
    batch = x.shape[0]
    bias2d = bias.reshape(1, n_classes)
    out = pl.pallas_call(
        _cwr_body,
        grid=(1,),
        in_specs=[
            pl.BlockSpec((batch, in_features), lambda i: (0, 0)),
            pl.BlockSpec((1, PAD_C), lambda i: (0, 0)),
            pl.BlockSpec(memory_space=pltpu.MemorySpace.HBM),
        ],
        out_specs=pl.BlockSpec((batch, PAD_C), lambda i: (0, 0)),
        out_shape=jax.ShapeDtypeStruct((batch, n_classes), jnp.float32),
        scratch_shapes=[
            pltpu.MemorySpace.VMEM((DEPTH, BLOCK_C, in_features), jnp.float32),
            pltpu.SemaphoreType.DMA((DEPTH,)),
        ],
    )(x, bias2d, weight)
    return out


# speedup vs baseline: 1.0256x; 1.0256x over previous
import jax
import jax.numpy as jnp
from jax.experimental import pallas as pl
from jax.experimental.pallas import tpu as pltpu

BLOCK_C = 12800
N_BLOCKS = 8
DEPTH = 4
PAD_C = BLOCK_C * N_BLOCKS


def _sizes(n_classes):
    sizes = []
    for i in range(N_BLOCKS):
        sizes.append(min(BLOCK_C, n_classes - i * BLOCK_C))
    return sizes


def _copies(i, w_hbm, wbuf, sems, size):
    slot = i % DEPTH
    half = size // 2
    parts = [(0, half), (half, size - half)]
    out = []
    for p, (off, sz) in enumerate(parts):
        out.append(
            pltpu.make_async_copy(
                w_hbm.at[pl.ds(i * BLOCK_C + off, sz)],
                wbuf.at[slot, pl.ds(off, sz)],
                sems.at[slot, p],
            )
        )
    return out


def _cwr_body(x_ref, b_ref, w_hbm, o_ref, wbuf, sems):
    n_classes = w_hbm.shape[0]
    sizes = _sizes(n_classes)

    def issue(i):
        for c in _copies(i, w_hbm, wbuf, sems, sizes[i]):
            c.start()

    for i in range(min(DEPTH, N_BLOCKS)):
        issue(i)

    x = x_ref[...]
    for i in range(N_BLOCKS):
        slot = i % DEPTH
        for c in _copies(i, w_hbm, wbuf, sems, sizes[i]):
            c.wait()
        acc = jax.lax.dot_general(
            x,
            wbuf[slot],
            (((1,), (1,)), ((), ())),
            preferred_element_type=jnp.float32,
        )
        sl = slice(i * BLOCK_C, (i + 1) * BLOCK_C)
        o_ref[:, sl] = acc + b_ref[:, sl]
        if i + DEPTH < N_BLOCKS:
            issue(i + DEPTH)


@jax.jit
def kernel(x, weight, bias):
    n_classes, in_features = weight.shape
    batch = x.shape[0]
    bias2d = bias.reshape(1, n_classes)
    out = pl.pallas_call(
        _cwr_body,
        grid=(1,),
        in_specs=[
            pl.BlockSpec((batch, in_features), lambda i: (0, 0)),
            pl.BlockSpec((1, PAD_C), lambda i: (0, 0)),
            pl.BlockSpec(memory_space=pltpu.MemorySpace.HBM),
        ],
        out_specs=pl.BlockSpec((batch, PAD_C), lambda i: (0, 0)),
        out_shape=jax.ShapeDtypeStruct((batch, n_classes), jnp.float32),
        scratch_shapes=[
            pltpu.MemorySpace.VMEM((DEPTH, BLOCK_C, in_features), jnp.float32),
            pltpu.SemaphoreType.DMA((DEPTH, 2)),
        ],
    )(x, bias2d, weight)
    return out


# speedup vs baseline: 1.0541x; 1.0278x over previous
import jax
import jax.numpy as jnp
from jax.experimental import pallas as pl
from jax.experimental.pallas import tpu as pltpu

BLOCK_C = 20096


def _linear_block(x_ref, w_ref, b_ref, o_ref):
    acc = jax.lax.dot_general(
        x_ref[...],
        w_ref[...],
        (((1,), (1,)), ((), ())),
        preferred_element_type=jnp.float32,
        precision=jax.lax.Precision.DEFAULT,
    )
    o_ref[...] = acc + b_ref[...]


@jax.jit
def kernel(x, weight, bias):
    n_classes, in_features = weight.shape
    batch = x.shape[0]
    bias2d = bias.reshape(1, n_classes)
    grid = (pl.cdiv(n_classes, BLOCK_C),)
    out = pl.pallas_call(
        _linear_block,
        grid=grid,
        in_specs=[
            pl.BlockSpec((batch, in_features), lambda i: (0, 0)),
            pl.BlockSpec((BLOCK_C, in_features), lambda i: (i, 0)),
            pl.BlockSpec((1, BLOCK_C), lambda i: (0, i)),
        ],
        out_specs=pl.BlockSpec((batch, BLOCK_C), lambda i: (0, i)),
        out_shape=jax.ShapeDtypeStruct((batch, n_classes), jnp.float32),
        compiler_params=pltpu.CompilerParams(
            dimension_semantics=("parallel",),
        ),
    )(x, weight, bias2d)
    return out
